# baseline (speedup 1.0000x reference)
import dataclasses
import functools

import jax
import jax.numpy as jnp
from jax import lax
from jax.experimental import pallas as pl
from jax.experimental.pallas import tpu as pltpu
from jax.experimental.pallas import tpu_sc as plsc

_NUM_CORES = 2
_NUM_SUBCORES = 16
_LANES = 16
_NW = _NUM_CORES * _NUM_SUBCORES

_W_BI = 0.3
_W_TRI = 0.15
_W_FOUR = 0.1

_K = 2
_UNROLL = 8


def kernel(input_ids, bigram_table, trigram_table, fourgram_table):
    b, t = input_ids.shape
    n = b * t
    v = bigram_table.shape[1]
    tri_buckets = trigram_table.shape[0]
    four_buckets = fourgram_table.shape[0]
    assert n % _NW == 0 and v % (_LANES * _UNROLL) == 0
    b_per_w = n // _NW

    flat = input_ids.reshape(-1).astype(jnp.int32)
    zero1 = jnp.zeros((1,), jnp.int32)
    prev2 = jnp.concatenate([zero1, flat[:-1]])
    prev3 = jnp.concatenate([zero1, zero1, flat[:-2]])

    def _bucket_fold(x, buckets):
        if buckets & (buckets - 1) == 0:
            return lax.bitwise_and(x, buckets - 1)
        return lax.rem(x, buckets)

    mesh = plsc.VectorSubcoreMesh(core_axis_name="c", subcore_axis_name="s")

    compiler_params = pltpu.CompilerParams()
    if "needs_layout_passes" in pltpu.CompilerParams.__dataclass_fields__:
        compiler_params = dataclasses.replace(
            compiler_params, needs_layout_passes=False)

    @functools.partial(
        pl.kernel,
        out_type=jax.ShapeDtypeStruct((n, v), jnp.float32),
        mesh=mesh,
        compiler_params=compiler_params,
        scratch_types=[
            pltpu.VMEM((b_per_w,), jnp.int32),
            pltpu.VMEM((b_per_w,), jnp.int32),
            pltpu.VMEM((b_per_w,), jnp.int32),
            pltpu.VMEM((b_per_w // _K * 8,), jnp.int32),
            pltpu.VMEM((b_per_w // _K * 8,), jnp.int32),
            pltpu.VMEM((b_per_w // _K * 8,), jnp.int32),
            pltpu.VMEM((_K, v), jnp.float32),
            pltpu.VMEM((_K, v), jnp.float32),
            pltpu.VMEM((_K, v), jnp.float32),
            pltpu.VMEM((_K, v), jnp.float32),
            pltpu.SemaphoreType.DMA,
        ],
    )
    def sc_kernel(p1_hbm, p2_hbm, p3_hbm, bi_hbm, tri_hbm, four_hbm, out_hbm,
                  in1, in2, in3, idx1, idx3, idx4, buf_b, buf_t, buf_f, obuf,
                  sem):
        wid = lax.axis_index("s") * _NUM_CORES + lax.axis_index("c")
        base = wid * b_per_w

        pltpu.sync_copy(p1_hbm.at[pl.ds(base, b_per_w)], in1)
        pltpu.sync_copy(p2_hbm.at[pl.ds(base, b_per_w)], in2)
        pltpu.sync_copy(p3_hbm.at[pl.ds(base, b_per_w)], in3)

        lane = lax.iota(jnp.int32, _LANES)
        slot = (lane // _K) * 8 + lax.rem(lane, _K)

        @pl.loop(0, b_per_w, step=_LANES)
        def _(i):
            s = pl.ds(i, _LANES)
            p1 = in1[s]
            p2 = in2[s]
            p3 = in3[s]
            partial_hash = 36313 * p1 + 27191 * p2
            h3 = _bucket_fold(partial_hash, tri_buckets)
            h4 = _bucket_fold(partial_hash + 51497 * p3, four_buckets)
            pos = slot + (i // _K) * 8
            plsc.store_scatter(idx1, [pos], p1)
            plsc.store_scatter(idx3, [pos], h3)
            plsc.store_scatter(idx4, [pos], h4)

        @pl.loop(0, b_per_w, step=_K)
        def _(g):
            go = g // _K * 8
            pltpu.async_copy(bi_hbm.at[idx1.at[pl.ds(go, _K)]], buf_b, sem).wait()
            pltpu.async_copy(tri_hbm.at[idx3.at[pl.ds(go, _K)]], buf_t, sem).wait()
            pltpu.async_copy(four_hbm.at[idx4.at[pl.ds(go, _K)]], buf_f, sem).wait()

            @pl.loop(0, _K)
            def _(r):
                @pl.loop(0, v, step=_LANES * _UNROLL)
                def _(c):
                    for u in range(_UNROLL):
                        cs = pl.ds(c + u * _LANES, _LANES)
                        obuf[r, cs] = (_W_BI * buf_b[r, cs]
                                       + _W_TRI * buf_t[r, cs]
                                       + _W_FOUR * buf_f[r, cs])

            pltpu.sync_copy(obuf, out_hbm.at[pl.ds(base + g, _K)])

    return sc_kernel(flat, prev2, prev3, bigram_table, trigram_table,
                     fourgram_table)


# speedup vs baseline: 1.5521x; 1.5521x over previous
import dataclasses
import functools

import jax
import jax.numpy as jnp
from jax import lax
from jax.experimental import pallas as pl
from jax.experimental.pallas import tpu as pltpu
from jax.experimental.pallas import tpu_sc as plsc

_NUM_CORES = 2
_NUM_SUBCORES = 16
_LANES = 16
_NW = _NUM_CORES * _NUM_SUBCORES

_W_BI = 0.3
_W_TRI = 0.15
_W_FOUR = 0.1

_K = 2
_UNROLL = 8


def kernel(input_ids, bigram_table, trigram_table, fourgram_table):
    b, t = input_ids.shape
    n = b * t
    v = bigram_table.shape[1]
    tri_buckets = trigram_table.shape[0]
    four_buckets = fourgram_table.shape[0]
    assert n % (_NW * _K) == 0 and v % (_LANES * _UNROLL) == 0
    b_per_w = n // _NW
    n_chunks = b_per_w // _K
    assert n_chunks % 2 == 0

    flat = input_ids.reshape(-1).astype(jnp.int32)
    zero1 = jnp.zeros((1,), jnp.int32)
    prev2 = jnp.concatenate([zero1, flat[:-1]])
    prev3 = jnp.concatenate([zero1, zero1, flat[:-2]])

    def _bucket_fold(x, buckets):
        if buckets & (buckets - 1) == 0:
            return lax.bitwise_and(x, buckets - 1)
        return lax.rem(x, buckets)

    mesh = plsc.VectorSubcoreMesh(core_axis_name="c", subcore_axis_name="s")

    compiler_params = pltpu.CompilerParams()
    if "needs_layout_passes" in pltpu.CompilerParams.__dataclass_fields__:
        compiler_params = dataclasses.replace(
            compiler_params, needs_layout_passes=False)

    @functools.partial(
        pl.kernel,
        out_type=jax.ShapeDtypeStruct((n, v), jnp.float32),
        mesh=mesh,
        compiler_params=compiler_params,
        scratch_types=[
            pltpu.VMEM((b_per_w,), jnp.int32),
            pltpu.VMEM((b_per_w,), jnp.int32),
            pltpu.VMEM((b_per_w,), jnp.int32),
            pltpu.VMEM((n_chunks * 8,), jnp.int32),
            pltpu.VMEM((n_chunks * 8,), jnp.int32),
            pltpu.VMEM((n_chunks * 8,), jnp.int32),
            pltpu.VMEM((2, _K, v), jnp.float32),
            pltpu.VMEM((2, _K, v), jnp.float32),
            pltpu.VMEM((2, _K, v), jnp.float32),
            pltpu.SemaphoreType.DMA,
            pltpu.SemaphoreType.DMA,
            pltpu.SemaphoreType.DMA,
            pltpu.SemaphoreType.DMA,
        ],
    )
    def sc_kernel(p1_hbm, p2_hbm, p3_hbm, bi_hbm, tri_hbm, four_hbm, out_hbm,
                  in1, in2, in3, idx1, idx3, idx4, buf_b, buf_t, buf_f,
                  gsem0, gsem1, osem0, osem1):
        wid = lax.axis_index("s") * _NUM_CORES + lax.axis_index("c")
        base = wid * b_per_w
        gsems = (gsem0, gsem1)
        osems = (osem0, osem1)

        pltpu.sync_copy(p1_hbm.at[pl.ds(base, b_per_w)], in1)
        pltpu.sync_copy(p2_hbm.at[pl.ds(base, b_per_w)], in2)
        pltpu.sync_copy(p3_hbm.at[pl.ds(base, b_per_w)], in3)

        lane = lax.iota(jnp.int32, _LANES)
        slot = (lane // _K) * 8 + lax.rem(lane, _K)

        @pl.loop(0, b_per_w, step=_LANES)
        def _(i):
            s = pl.ds(i, _LANES)
            p1 = in1[s]
            p2 = in2[s]
            p3 = in3[s]
            partial_hash = 36313 * p1 + 27191 * p2
            h3 = _bucket_fold(partial_hash, tri_buckets)
            h4 = _bucket_fold(partial_hash + 51497 * p3, four_buckets)
            pos = slot + (i // _K) * 8
            plsc.store_scatter(idx1, [pos], p1)
            plsc.store_scatter(idx3, [pos], h3)
            plsc.store_scatter(idx4, [pos], h4)

        def gather_copies(g, si):
            go = g * 8
            return (
                pltpu.make_async_copy(
                    bi_hbm.at[idx1.at[pl.ds(go, _K)]], buf_b.at[si], gsems[si]),
                pltpu.make_async_copy(
                    tri_hbm.at[idx3.at[pl.ds(go, _K)]], buf_t.at[si], gsems[si]),
                pltpu.make_async_copy(
                    four_hbm.at[idx4.at[pl.ds(go, _K)]], buf_f.at[si], gsems[si]),
            )

        def out_copy(g, si):
            return pltpu.make_async_copy(
                buf_b.at[si], out_hbm.at[pl.ds(base + g * _K, _K)], osems[si])

        def issue_gathers(g, si):
            for c in gather_copies(g, si):
                c.start()

        def wait_gathers(g, si):
            for c in gather_copies(g, si):
                c.wait()

        def compute(si):
            @pl.loop(0, _K)
            def _(r):
                @pl.loop(0, v, step=_LANES * _UNROLL)
                def _(c):
                    for u in range(_UNROLL):
                        cs = pl.ds(c + u * _LANES, _LANES)
                        buf_b[si, r, cs] = (_W_BI * buf_b[si, r, cs]
                                            + _W_TRI * buf_t[si, r, cs]
                                            + _W_FOUR * buf_f[si, r, cs])

        issue_gathers(0, 0)

        @pl.loop(0, n_chunks, step=2)
        def _(g):
            issue_gathers(g + 1, 1)
            wait_gathers(g, 0)

            @pl.when(g >= 2)
            def _():
                out_copy(g - 2, 0).wait()

            compute(0)
            out_copy(g, 0).start()

            @pl.when(g + 2 < n_chunks)
            def _():
                issue_gathers(g + 2, 0)

            wait_gathers(g + 1, 1)

            @pl.when(g >= 2)
            def _():
                out_copy(g - 1, 1).wait()

            compute(1)
            out_copy(g + 1, 1).start()

        out_copy(n_chunks - 2, 0).wait()
        out_copy(n_chunks - 1, 1).wait()

    return sc_kernel(flat, prev2, prev3, bigram_table, trigram_table,
                     fourgram_table)


# speedup vs baseline: 4.1184x; 2.6535x over previous
import dataclasses
import functools

import jax
import jax.numpy as jnp
from jax import lax
from jax.experimental import pallas as pl
from jax.experimental.pallas import tpu as pltpu
from jax.experimental.pallas import tpu_sc as plsc

_NUM_CORES = 2
_NUM_SUBCORES = 16
_LANES = 16
_NW = _NUM_CORES * _NUM_SUBCORES

_W_BI = 0.3
_W_TRI = 0.15
_W_FOUR = 0.1

_DO_COMPUTE = True
_K = 2
_UNROLL = 8


def kernel(input_ids, bigram_table, trigram_table, fourgram_table):
    b, t = input_ids.shape
    n = b * t
    v = bigram_table.shape[1]
    tri_buckets = trigram_table.shape[0]
    four_buckets = fourgram_table.shape[0]
    assert n % (_NW * _K) == 0 and v % (_LANES * _UNROLL) == 0
    b_per_w = n // _NW
    n_chunks = b_per_w // _K
    assert n_chunks % 2 == 0

    flat = input_ids.reshape(-1).astype(jnp.int32)
    zero1 = jnp.zeros((1,), jnp.int32)
    prev2 = jnp.concatenate([zero1, flat[:-1]])
    prev3 = jnp.concatenate([zero1, zero1, flat[:-2]])

    def _bucket_fold(x, buckets):
        if buckets & (buckets - 1) == 0:
            return lax.bitwise_and(x, buckets - 1)
        return lax.rem(x, buckets)

    mesh = plsc.VectorSubcoreMesh(core_axis_name="c", subcore_axis_name="s")

    compiler_params = pltpu.CompilerParams()
    if "needs_layout_passes" in pltpu.CompilerParams.__dataclass_fields__:
        compiler_params = dataclasses.replace(
            compiler_params, needs_layout_passes=False)

    @functools.partial(
        pl.kernel,
        out_type=jax.ShapeDtypeStruct((n, v), jnp.float32),
        mesh=mesh,
        compiler_params=compiler_params,
        scratch_types=[
            pltpu.VMEM((b_per_w,), jnp.int32),
            pltpu.VMEM((b_per_w,), jnp.int32),
            pltpu.VMEM((b_per_w,), jnp.int32),
            pltpu.VMEM((n_chunks * 8,), jnp.int32),
            pltpu.VMEM((n_chunks * 8,), jnp.int32),
            pltpu.VMEM((n_chunks * 8,), jnp.int32),
            pltpu.VMEM((2, _K, v), jnp.float32),
            pltpu.VMEM((2, _K, v), jnp.float32),
            pltpu.VMEM((2, _K, v), jnp.float32),
            pltpu.SemaphoreType.DMA,
            pltpu.SemaphoreType.DMA,
            pltpu.SemaphoreType.DMA,
            pltpu.SemaphoreType.DMA,
        ],
    )
    def sc_kernel(p1_hbm, p2_hbm, p3_hbm, bi_hbm, tri_hbm, four_hbm, out_hbm,
                  in1, in2, in3, idx1, idx3, idx4, buf_b, buf_t, buf_f,
                  gsem0, gsem1, osem0, osem1):
        wid = lax.axis_index("s") * _NUM_CORES + lax.axis_index("c")
        base = wid * b_per_w
        gsems = (gsem0, gsem1)
        osems = (osem0, osem1)

        pltpu.sync_copy(p1_hbm.at[pl.ds(base, b_per_w)], in1)
        pltpu.sync_copy(p2_hbm.at[pl.ds(base, b_per_w)], in2)
        pltpu.sync_copy(p3_hbm.at[pl.ds(base, b_per_w)], in3)

        lane = lax.iota(jnp.int32, _LANES)
        slot = (lane // _K) * 8 + lax.rem(lane, _K)

        @pl.loop(0, b_per_w, step=_LANES)
        def _(i):
            s = pl.ds(i, _LANES)
            p1 = in1[s]
            p2 = in2[s]
            p3 = in3[s]
            partial_hash = 36313 * p1 + 27191 * p2
            h3 = _bucket_fold(partial_hash, tri_buckets)
            h4 = _bucket_fold(partial_hash + 51497 * p3, four_buckets)
            pos = slot + (i // _K) * 8
            plsc.store_scatter(idx1, [pos], p1)
            plsc.store_scatter(idx3, [pos], h3)
            plsc.store_scatter(idx4, [pos], h4)

        def gather_copies(g, si):
            go = g * 8
            return (
                pltpu.make_async_copy(
                    bi_hbm.at[idx1.at[pl.ds(go, _K)]], buf_b.at[si], gsems[si]),
                pltpu.make_async_copy(
                    tri_hbm.at[idx3.at[pl.ds(go, _K)]], buf_t.at[si], gsems[si]),
                pltpu.make_async_copy(
                    four_hbm.at[idx4.at[pl.ds(go, _K)]], buf_f.at[si], gsems[si]),
            )

        def out_copy(g, si):
            return pltpu.make_async_copy(
                buf_b.at[si], out_hbm.at[pl.ds(base + g * _K, _K)], osems[si])

        def issue_gathers(g, si):
            for c in gather_copies(g, si):
                c.start()

        def wait_gathers(g, si):
            for c in gather_copies(g, si):
                c.wait()

        def compute(si):
            for r in range(_K):
                @plsc.parallel_loop(0, v, step=_LANES, unroll=_UNROLL)
                def _(c):
                    cs = pl.ds(c, _LANES)
                    buf_b[si, r, cs] = (_W_BI * buf_b[si, r, cs]
                                        + _W_TRI * buf_t[si, r, cs]
                                        + _W_FOUR * buf_f[si, r, cs])

        issue_gathers(0, 0)

        @pl.loop(0, n_chunks, step=2)
        def _(g):
            issue_gathers(g + 1, 1)
            wait_gathers(g, 0)

            @pl.when(g >= 2)
            def _():
                out_copy(g - 2, 0).wait()

            if _DO_COMPUTE:
                compute(0)
            out_copy(g, 0).start()

            @pl.when(g + 2 < n_chunks)
            def _():
                issue_gathers(g + 2, 0)

            wait_gathers(g + 1, 1)

            @pl.when(g >= 2)
            def _():
                out_copy(g - 1, 1).wait()

            if _DO_COMPUTE:
                compute(1)
            out_copy(g + 1, 1).start()

        out_copy(n_chunks - 2, 0).wait()
        out_copy(n_chunks - 1, 1).wait()

    return sc_kernel(flat, prev2, prev3, bigram_table, trigram_table,
                     fourgram_table)
